# baseline (speedup 1.0000x reference)
import functools

import jax
import jax.numpy as jnp
from jax import lax
from jax.experimental import pallas as pl
from jax.experimental.pallas import tpu as pltpu
from jax.experimental.pallas import tpu_sc as plsc

B = 16384
D = 64
NC = 2
NS = 16
NW = NC * NS
BPW = B // NW

@functools.cache
def _make_sc_gather():
    mesh = plsc.VectorSubcoreMesh(
        core_axis_name="c", subcore_axis_name="s",
        num_cores=NC, num_subcores=NS)

    @functools.partial(
        pl.kernel,
        out_type=(
            jax.ShapeDtypeStruct((B, D), jnp.float32),
            jax.ShapeDtypeStruct((B, D), jnp.float32),
        ),
        mesh=mesh,
        compiler_params=pltpu.CompilerParams(use_tc_tiling_on_sc=False),
        scratch_types=[
            pltpu.VMEM((BPW,), jnp.int32),
            pltpu.VMEM((BPW,), jnp.int32),
            pltpu.VMEM((BPW, D), jnp.float32),
            pltpu.VMEM((BPW, D), jnp.float32),
            pltpu.SemaphoreType.DMA,
            pltpu.SemaphoreType.DMA,
        ],
    )
    def _sc_gather(u_hbm, i_hbm, uemb_hbm, iemb_hbm, gu_hbm, gi_hbm,
                   uidx_v, iidx_v, urows_v, irows_v, sem_u, sem_i):
        wid = lax.axis_index("s") * NC + lax.axis_index("c")
        base = wid * BPW
        pltpu.sync_copy(u_hbm.at[pl.ds(base, BPW)], uidx_v)
        pltpu.sync_copy(i_hbm.at[pl.ds(base, BPW)], iidx_v)
        cp_u = pltpu.async_copy(uemb_hbm.at[uidx_v], urows_v, sem_u)
        cp_i = pltpu.async_copy(iemb_hbm.at[iidx_v], irows_v, sem_i)
        cp_u.wait()
        pltpu.sync_copy(urows_v, gu_hbm.at[pl.ds(base, BPW)])
        cp_i.wait()
        pltpu.sync_copy(irows_v, gi_hbm.at[pl.ds(base, BPW)])

    return _sc_gather


BK = 2048


def _mlp_body(gu, gi, ua, ia, Wu, Wi, bu, bi, W1, b1, W2, out):
    cdims = (((1,), (1,)), ((), ()))
    u_e = gu[...] + lax.dot_general(ua[...], Wu[...], cdims,
                                    preferred_element_type=jnp.float32) + bu[...]
    i_e = gi[...] + lax.dot_general(ia[...], Wi[...], cdims,
                                    preferred_element_type=jnp.float32) + bi[...]
    w1 = W1[...]
    h = lax.dot_general(u_e, w1[:, :D], cdims,
                        preferred_element_type=jnp.float32)
    h = h + lax.dot_general(i_e, w1[:, D:], cdims,
                            preferred_element_type=jnp.float32)
    h = jnp.maximum(h + b1[...], 0.0)
    out[...] = lax.dot_general(h, W2[...], cdims,
                               preferred_element_type=jnp.float32)


def kernel(u, i, ua, ia, user_emb, item_emb, Wu, bu, Wi, bi, W1, b1, W2, b2):
    gu, gi = _make_sc_gather()(u, i, user_emb, item_emb)

    grid = (B // BK,)
    blk = lambda c: pl.BlockSpec((BK, c), lambda g: (g, 0))
    full = lambda shape: pl.BlockSpec(shape, lambda g: (0,) * len(shape))
    out = pl.pallas_call(
        _mlp_body,
        grid=grid,
        in_specs=[
            blk(D), blk(D),
            blk(128), blk(128),
            full((D, 128)), full((D, 128)),
            full((1, D)), full((1, D)),
            full((128, 128)), full((1, 128)),
            full((1, 128)),
        ],
        out_specs=pl.BlockSpec((BK, 1), lambda g: (g, 0)),
        out_shape=jax.ShapeDtypeStruct((B, 1), jnp.float32),
    )(gu, gi, ua, ia, Wu, Wi,
      bu.reshape(1, D), bi.reshape(1, D), W1, b1.reshape(1, 128), W2)
    return out.reshape(B) + b2[0]


# speedup vs baseline: 1.6479x; 1.6479x over previous
import functools

import jax
import jax.numpy as jnp
from jax import lax
from jax.experimental import pallas as pl
from jax.experimental.pallas import tpu as pltpu
from jax.experimental.pallas import tpu_sc as plsc

B = 16384
D = 64
NC = 2
NS = 16
NW = NC * NS
BPW = B // NW

@functools.cache
def _make_sc_gather():
    mesh = plsc.VectorSubcoreMesh(
        core_axis_name="c", subcore_axis_name="s",
        num_cores=NC, num_subcores=NS)

    @functools.partial(
        pl.kernel,
        out_type=jax.ShapeDtypeStruct((B, 2 * D), jnp.float32),
        mesh=mesh,
        scratch_types=[
            pltpu.VMEM((BPW,), jnp.int32),
            pltpu.VMEM((BPW,), jnp.int32),
            pltpu.VMEM((BPW,), jnp.int32),
            pltpu.VMEM((BPW, 2 * D), jnp.float32),
            pltpu.SemaphoreType.DMA,
            pltpu.SemaphoreType.DMA,
        ],
    )
    def _sc_gather(u_hbm, i_hbm, uemb_hbm, iemb_hbm, x_hbm,
                   uidx_v, iidx_v, oidx_v, x_v, sem_g, sem_o):
        wid = lax.axis_index("s") * NC + lax.axis_index("c")
        base = wid * BPW
        pltpu.sync_copy(u_hbm.at[pl.ds(base, BPW)], uidx_v)
        pltpu.sync_copy(i_hbm.at[pl.ds(base, BPW)], iidx_v)
        lanes = lax.iota(jnp.int32, 16)

        def issue(g, _):
            base16 = g * 16
            vu = uidx_v[pl.ds(base16, 16)]
            vi = iidx_v[pl.ds(base16, 16)]
            oidx_v[pl.ds(base16, 16)] = base + base16 + lanes
            for k in range(16):
                pltpu.async_copy(
                    uemb_hbm.at[vu[k]], x_v.at[base16 + k, pl.ds(0, D)],
                    sem_g)
                pltpu.async_copy(
                    iemb_hbm.at[vi[k]], x_v.at[base16 + k, pl.ds(D, D)],
                    sem_g)
            return _

        lax.fori_loop(0, BPW // 16, issue, 0, unroll=False)
        pltpu.make_async_copy(
            x_hbm.at[pl.ds(0, BPW)], x_v, sem_g).wait()
        pltpu.async_copy(x_v, x_hbm.at[oidx_v], sem_o).wait()

    return _sc_gather


BK = 2048


def _mlp_body(xg, ua, ia, Wu, Wi, bu, bi, W1, b1, W2, out):
    cdims = (((1,), (1,)), ((), ()))
    x = xg[...]
    u_e = x[:, :D] + lax.dot_general(ua[...], Wu[...], cdims,
                                    preferred_element_type=jnp.float32) + bu[...]
    i_e = x[:, D:] + lax.dot_general(ia[...], Wi[...], cdims,
                                    preferred_element_type=jnp.float32) + bi[...]
    w1 = W1[...]
    h = lax.dot_general(u_e, w1[:, :D], cdims,
                        preferred_element_type=jnp.float32)
    h = h + lax.dot_general(i_e, w1[:, D:], cdims,
                            preferred_element_type=jnp.float32)
    h = jnp.maximum(h + b1[...], 0.0)
    out[...] = lax.dot_general(h, W2[...], cdims,
                               preferred_element_type=jnp.float32)


def kernel(u, i, ua, ia, user_emb, item_emb, Wu, bu, Wi, bi, W1, b1, W2, b2):
    xg = _make_sc_gather()(u, i, user_emb, item_emb)

    grid = (B // BK,)
    blk = lambda c: pl.BlockSpec((BK, c), lambda g: (g, 0))
    full = lambda shape: pl.BlockSpec(shape, lambda g: (0,) * len(shape))
    out = pl.pallas_call(
        _mlp_body,
        grid=grid,
        in_specs=[
            blk(2 * D),
            blk(128), blk(128),
            full((D, 128)), full((D, 128)),
            full((1, D)), full((1, D)),
            full((128, 128)), full((1, 128)),
            full((1, 128)),
        ],
        out_specs=pl.BlockSpec((BK, 1), lambda g: (g, 0)),
        out_shape=jax.ShapeDtypeStruct((B, 1), jnp.float32),
    )(xg, ua, ia, Wu, Wi,
      bu.reshape(1, D), bi.reshape(1, D), W1, b1.reshape(1, 128), W2)
    return out.reshape(B) + b2[0]
